# baseline (speedup 1.0000x reference)
import functools

import jax
import jax.numpy as jnp
from jax import lax
from jax.experimental import pallas as pl
from jax.experimental.pallas import tpu as pltpu
from jax.experimental.pallas import tpu_sc as plsc

ITEM_NUM = 1000000
EMB = 64
MAX_LEN = 200
BATCH = 4096

_INFO = plsc.get_sparse_core_info()
_NC = _INFO.num_cores
_NS = _INFO.num_subcores
_NW = _NC * _NS
_L = 16

_PAD_ITEMS = 1000064
_W = 80
_TCOLS = _PAD_ITEMS // 128
_K1_ITERS = (_TCOLS + _NW - 1) // _NW


def _make_k1():
    mesh = plsc.VectorSubcoreMesh(core_axis_name="c", subcore_axis_name="s")

    @functools.partial(
        pl.kernel,
        out_type=jax.ShapeDtypeStruct((_PAD_ITEMS * _W,), jnp.float32),
        mesh=mesh,
        scratch_types=[
            pltpu.VMEM((EMB, 128), jnp.float32),
            pltpu.VMEM((EMB, 128), jnp.float32),
            pltpu.VMEM((128 * _W,), jnp.float32),
            pltpu.SemaphoreType.DMA,
            pltpu.SemaphoreType.DMA,
        ],
        compiler_params=pltpu.CompilerParams(
            disable_bounds_checks=True, needs_layout_passes=False),
    )
    def k1(tabT_hbm, out_hbm, stg_a, stg_b, outv, sem_a, sem_b):
        wid = lax.axis_index("s") * _NC + lax.axis_index("c")
        tok_w = [(lax.iota(jnp.int32, _L) + _L * t) * _W for t in range(128 // _L)]

        def stage_start(c, buf, sem):
            pltpu.async_copy(tabT_hbm.at[:, pl.ds(c * 128, 128)], buf, sem)

        def stage_wait(c, buf, sem):
            pltpu.make_async_copy(tabT_hbm.at[:, pl.ds(c * 128, 128)],
                                  buf, sem).wait()

        def transpose_out(c, buf):
            @plsc.parallel_loop(0, EMB, unroll=8)
            def tbody(d):
                dv = jnp.full((_L,), d, jnp.int32)
                for t in range(128 // _L):
                    v = buf[d, pl.ds(_L * t, _L)]
                    plsc.store_scatter(outv, [tok_w[t] + dv], v)
            pltpu.sync_copy(outv, out_hbm.at[pl.ds(c * (128 * _W), 128 * _W)])

        bufs = (stg_a, stg_b)
        sems = (sem_a, sem_b)

        stage_start(wid, stg_a, sem_a)

        def iters(k, carry):
            c = wid + _NW * k
            nxt = c + _NW
            for par in range(2):
                @pl.when((k % 2 == par) & (c < _TCOLS))
                def _():
                    @pl.when(nxt < _TCOLS)
                    def _():
                        stage_start(nxt, bufs[1 - par], sems[1 - par])
                    stage_wait(c, bufs[par], sems[par])
                    transpose_out(c, bufs[par])
            return carry

        lax.fori_loop(0, _K1_ITERS, iters, 0)

    return k1


def _make_k2():
    mesh = plsc.VectorSubcoreMesh(core_axis_name="c", subcore_axis_name="s")

    @functools.partial(
        pl.kernel,
        out_type=jax.ShapeDtypeStruct((MAX_LEN, 8, 32, 8, 128), jnp.float32),
        mesh=mesh,
        scratch_types=[
            pltpu.VMEM((MAX_LEN, 128), jnp.int32),
            pltpu.VMEM((128, _W), jnp.float32),
            pltpu.VMEM((128, _W), jnp.float32),
            pltpu.VMEM((EMB, _L), jnp.float32),
            pltpu.VMEM((EMB, _L), jnp.float32),
            pltpu.VMEM((8, 8, 128), jnp.float32),
            pltpu.VMEM((8, 8, 128), jnp.float32),
            pltpu.SemaphoreType.DMA,
            pltpu.SemaphoreType.DMA,
            pltpu.SemaphoreType.DMA,
            pltpu.SemaphoreType.DMA,
            pltpu.SemaphoreType.DMA,
            pltpu.SemaphoreType.DMA,
        ],
        compiler_params=pltpu.CompilerParams(
            use_tc_tiling_on_sc=False, needs_layout_passes=False),
    )
    def k2(tab_hbm, seqsT_hbm, posrep_hbm, out_hbm,
           idxv, rows_a, rows_b, pos_a, pos_b, til_a, til_b,
           gsa, gsb, psa, psb, osa, osb):
        wid = lax.axis_index("s") * _NC + lax.axis_index("c")
        row16 = [lax.iota(jnp.int32, _L) + _L * t for t in range(128 // _L)]

        pltpu.sync_copy(seqsT_hbm.at[:, pl.ds(wid * 128, 128)], idxv)

        def fetch_start(l, rows, gsem, posb, psem):
            pltpu.async_copy(tab_hbm.at[idxv.at[l]], rows, gsem)
            pltpu.async_copy(posrep_hbm.at[l], posb, psem)

        def fetch_wait(l, rows, gsem, posb, psem):
            pltpu.make_async_copy(tab_hbm.at[idxv.at[l]], rows, gsem).wait()
            pltpu.make_async_copy(posrep_hbm.at[l], posb, psem).wait()

        def out_start(l, tiles, sem):
            pltpu.async_copy(tiles, out_hbm.at[l, :, wid], sem)

        def out_wait(l, tiles, sem):
            pltpu.make_async_copy(tiles, out_hbm.at[l, :, wid], sem).wait()

        def compute(rows, posb, tiles):
            @plsc.parallel_loop(0, EMB, unroll=8)
            def cbody(d):
                r = d >> 3
                s = d & 7
                pv = posb[d, :]
                dv = jnp.full((_L,), d, jnp.int32)
                for t in range(128 // _L):
                    v = plsc.load_gather(rows, [row16[t], dv])
                    tiles[r, s, pl.ds(_L * t, _L)] = v + pv

        fetch_start(0, rows_a, gsa, pos_a, psa)

        def body(i, carry):
            l0 = 2 * i
            l1 = l0 + 1

            @pl.when(i > 0)
            def _():
                out_wait(l1 - 2, til_b, osb)
            fetch_start(l1, rows_b, gsb, pos_b, psb)

            fetch_wait(l0, rows_a, gsa, pos_a, psa)
            compute(rows_a, pos_a, til_a)
            out_start(l0, til_a, osa)

            @pl.when(i < MAX_LEN // 2 - 1)
            def _():
                out_wait(l0, til_a, osa)
                fetch_start(l0 + 2, rows_a, gsa, pos_a, psa)

            fetch_wait(l1, rows_b, gsb, pos_b, psb)
            compute(rows_b, pos_b, til_b)
            out_start(l1, til_b, osb)
            return carry

        lax.fori_loop(0, MAX_LEN // 2, body, 0)
        out_wait(MAX_LEN - 2, til_a, osa)
        out_wait(MAX_LEN - 1, til_b, osb)

    return k2


_K1 = _make_k1()
_K2 = _make_k2()


def kernel(batch_seqs, token_table, pos_table):
    seqsT = batch_seqs.astype(jnp.int32).T
    posrep = jnp.broadcast_to(pos_table[:, :, None], (MAX_LEN, EMB, _L))
    lin = _K1(token_table.T)
    out5 = _K2(lin.reshape(_PAD_ITEMS, _W), seqsT, posrep)
    return out5.transpose(2, 4, 0, 1, 3).reshape(BATCH, MAX_LEN, EMB)


# speedup vs baseline: 1.1725x; 1.1725x over previous
import functools

import jax
import jax.numpy as jnp
from jax import lax
from jax.experimental import pallas as pl
from jax.experimental.pallas import tpu as pltpu
from jax.experimental.pallas import tpu_sc as plsc

ITEM_NUM = 1000000
EMB = 64
MAX_LEN = 200
BATCH = 4096

_INFO = plsc.get_sparse_core_info()
_NC = _INFO.num_cores
_NS = _INFO.num_subcores
_NW = _NC * _NS
_L = 16

_PAD_ITEMS = 1000064
_W = 80
_TCOLS = _PAD_ITEMS // 128
_K1_ITERS = (_TCOLS + _NW - 1) // _NW


def _make_k1():
    mesh = plsc.VectorSubcoreMesh(core_axis_name="c", subcore_axis_name="s")

    @functools.partial(
        pl.kernel,
        out_type=jax.ShapeDtypeStruct((_PAD_ITEMS * _W,), jnp.float32),
        mesh=mesh,
        scratch_types=[
            pltpu.VMEM((EMB, 128), jnp.float32),
            pltpu.VMEM((EMB, 128), jnp.float32),
            pltpu.VMEM((128 * _W,), jnp.float32),
            pltpu.SemaphoreType.DMA,
            pltpu.SemaphoreType.DMA,
        ],
        compiler_params=pltpu.CompilerParams(
            disable_bounds_checks=True, needs_layout_passes=False),
    )
    def k1(tabT_hbm, out_hbm, stg_a, stg_b, outv, sem_a, sem_b):
        wid = lax.axis_index("s") * _NC + lax.axis_index("c")
        tok_w = [(lax.iota(jnp.int32, _L) + _L * t) * _W for t in range(128 // _L)]

        def stage_start(c, buf, sem):
            pltpu.async_copy(tabT_hbm.at[:, pl.ds(c * 128, 128)], buf, sem)

        def stage_wait(c, buf, sem):
            pltpu.make_async_copy(tabT_hbm.at[:, pl.ds(c * 128, 128)],
                                  buf, sem).wait()

        def transpose_out(c, buf):
            @plsc.parallel_loop(0, EMB, unroll=4)
            def tbody(d):
                dv = jnp.full((_L,), d, jnp.int32)
                for t in range(128 // _L):
                    v = buf[d, pl.ds(_L * t, _L)]
                    plsc.store_scatter(outv, [tok_w[t] + dv], v)
            pltpu.sync_copy(outv, out_hbm.at[pl.ds(c * (128 * _W), 128 * _W)])

        bufs = (stg_a, stg_b)
        sems = (sem_a, sem_b)

        stage_start(wid, stg_a, sem_a)

        def iters(k, carry):
            c = wid + _NW * k
            nxt = c + _NW
            for par in range(2):
                @pl.when((k % 2 == par) & (c < _TCOLS))
                def _():
                    @pl.when(nxt < _TCOLS)
                    def _():
                        stage_start(nxt, bufs[1 - par], sems[1 - par])
                    stage_wait(c, bufs[par], sems[par])
                    transpose_out(c, bufs[par])
            return carry

        lax.fori_loop(0, _K1_ITERS, iters, 0)

    return k1


def _make_k2():
    mesh = plsc.VectorSubcoreMesh(core_axis_name="c", subcore_axis_name="s")

    @functools.partial(
        pl.kernel,
        out_type=jax.ShapeDtypeStruct((MAX_LEN, 8, 32, 8, 128), jnp.float32),
        mesh=mesh,
        scratch_types=[
            pltpu.VMEM((MAX_LEN, 128), jnp.int32),
            pltpu.VMEM((128, _W), jnp.float32),
            pltpu.VMEM((128, _W), jnp.float32),
            pltpu.VMEM((EMB, _L), jnp.float32),
            pltpu.VMEM((EMB, _L), jnp.float32),
            pltpu.VMEM((8, 8, 128), jnp.float32),
            pltpu.VMEM((8, 8, 128), jnp.float32),
            pltpu.SemaphoreType.DMA,
            pltpu.SemaphoreType.DMA,
            pltpu.SemaphoreType.DMA,
            pltpu.SemaphoreType.DMA,
            pltpu.SemaphoreType.DMA,
            pltpu.SemaphoreType.DMA,
        ],
        compiler_params=pltpu.CompilerParams(
            use_tc_tiling_on_sc=False, needs_layout_passes=False),
    )
    def k2(tab_hbm, seqsT_hbm, posrep_hbm, out_hbm,
           idxv, rows_a, rows_b, pos_a, pos_b, til_a, til_b,
           gsa, gsb, psa, psb, osa, osb):
        wid = lax.axis_index("s") * _NC + lax.axis_index("c")
        row16 = [lax.iota(jnp.int32, _L) + _L * t for t in range(128 // _L)]

        pltpu.sync_copy(seqsT_hbm.at[:, pl.ds(wid * 128, 128)], idxv)

        def fetch_start(l, rows, gsem, posb, psem):
            pltpu.async_copy(tab_hbm.at[idxv.at[l]], rows, gsem)
            pltpu.async_copy(posrep_hbm.at[l], posb, psem)

        def fetch_wait(l, rows, gsem, posb, psem):
            pltpu.make_async_copy(tab_hbm.at[idxv.at[l]], rows, gsem).wait()
            pltpu.make_async_copy(posrep_hbm.at[l], posb, psem).wait()

        def out_start(l, tiles, sem):
            pltpu.async_copy(tiles, out_hbm.at[l, :, wid], sem)

        def out_wait(l, tiles, sem):
            pltpu.make_async_copy(tiles, out_hbm.at[l, :, wid], sem).wait()

        def compute(rows, posb, tiles):
            @plsc.parallel_loop(0, EMB, unroll=4)
            def cbody(d):
                r = d >> 3
                s = d & 7
                pv = posb[d, :]
                dv = jnp.full((_L,), d, jnp.int32)
                for t in range(128 // _L):
                    v = plsc.load_gather(rows, [row16[t], dv])
                    tiles[r, s, pl.ds(_L * t, _L)] = v + pv

        fetch_start(0, rows_a, gsa, pos_a, psa)

        def body(i, carry):
            l0 = 2 * i
            l1 = l0 + 1

            @pl.when(i > 0)
            def _():
                out_wait(l1 - 2, til_b, osb)
            fetch_start(l1, rows_b, gsb, pos_b, psb)

            fetch_wait(l0, rows_a, gsa, pos_a, psa)
            compute(rows_a, pos_a, til_a)
            out_start(l0, til_a, osa)

            @pl.when(i < MAX_LEN // 2 - 1)
            def _():
                out_wait(l0, til_a, osa)
                fetch_start(l0 + 2, rows_a, gsa, pos_a, psa)

            fetch_wait(l1, rows_b, gsb, pos_b, psb)
            compute(rows_b, pos_b, til_b)
            out_start(l1, til_b, osb)
            return carry

        lax.fori_loop(0, MAX_LEN // 2, body, 0)
        out_wait(MAX_LEN - 2, til_a, osa)
        out_wait(MAX_LEN - 1, til_b, osb)

    return k2


_K1 = _make_k1()
_K2 = _make_k2()


def kernel(batch_seqs, token_table, pos_table):
    seqsT = batch_seqs.astype(jnp.int32).T
    posrep = jnp.broadcast_to(pos_table[:, :, None], (MAX_LEN, EMB, _L))
    lin = _K1(token_table.T)
    out5 = _K2(lin.reshape(_PAD_ITEMS, _W), seqsT, posrep)
    return out5.transpose(2, 4, 0, 1, 3).reshape(BATCH, MAX_LEN, EMB)


# speedup vs baseline: 1.3376x; 1.1408x over previous
import functools

import jax
import jax.numpy as jnp
from jax import lax
from jax.experimental import pallas as pl
from jax.experimental.pallas import tpu as pltpu
from jax.experimental.pallas import tpu_sc as plsc

ITEM_NUM = 1000000
EMB = 64
MAX_LEN = 200
BATCH = 4096

_INFO = plsc.get_sparse_core_info()
_NC = _INFO.num_cores
_NS = _INFO.num_subcores
_NW = _NC * _NS
_L = 16

_PAD_ITEMS = 1000064
_W = 80
_TCOLS = _PAD_ITEMS // 128
_K1_ITERS = (_TCOLS + _NW - 1) // _NW


def _make_k1():
    mesh = plsc.VectorSubcoreMesh(core_axis_name="c", subcore_axis_name="s")

    @functools.partial(
        pl.kernel,
        out_type=jax.ShapeDtypeStruct((_PAD_ITEMS * _W,), jnp.float32),
        mesh=mesh,
        scratch_types=[
            pltpu.VMEM((EMB, 128), jnp.float32),
            pltpu.VMEM((EMB, 128), jnp.float32),
            pltpu.VMEM((128 * _W,), jnp.float32),
            pltpu.SemaphoreType.DMA,
            pltpu.SemaphoreType.DMA,
        ],
        compiler_params=pltpu.CompilerParams(
            disable_bounds_checks=True, needs_layout_passes=False),
    )
    def k1(tabT_hbm, out_hbm, stg_a, stg_b, outv, sem_a, sem_b):
        wid = lax.axis_index("s") * _NC + lax.axis_index("c")
        tok_w = [(lax.iota(jnp.int32, _L) + _L * t) * _W for t in range(128 // _L)]

        def stage_start(c, buf, sem):
            pltpu.async_copy(tabT_hbm.at[:, pl.ds(c * 128, 128)], buf, sem)

        def stage_wait(c, buf, sem):
            pltpu.make_async_copy(tabT_hbm.at[:, pl.ds(c * 128, 128)],
                                  buf, sem).wait()

        def transpose_out(c, buf):
            @plsc.parallel_loop(0, EMB, unroll=4)
            def tbody(d):
                dv = jnp.full((_L,), d, jnp.int32)
                for t in range(128 // _L):
                    v = buf[d, pl.ds(_L * t, _L)]
                    plsc.store_scatter(outv, [tok_w[t] + dv], v)
            pltpu.sync_copy(outv, out_hbm.at[pl.ds(c * (128 * _W), 128 * _W)])

        bufs = (stg_a, stg_b)
        sems = (sem_a, sem_b)

        stage_start(wid, stg_a, sem_a)

        def iters(k, carry):
            c = wid + _NW * k
            nxt = c + _NW
            for par in range(2):
                @pl.when((k % 2 == par) & (c < _TCOLS))
                def _():
                    @pl.when(nxt < _TCOLS)
                    def _():
                        stage_start(nxt, bufs[1 - par], sems[1 - par])
                    stage_wait(c, bufs[par], sems[par])
                    transpose_out(c, bufs[par])
            return carry

        lax.fori_loop(0, _K1_ITERS, iters, 0)

    return k1


def _make_k2():
    mesh = plsc.VectorSubcoreMesh(core_axis_name="c", subcore_axis_name="s")

    @functools.partial(
        pl.kernel,
        out_type=jax.ShapeDtypeStruct((MAX_LEN, 8, 32, 8, 128), jnp.float32),
        mesh=mesh,
        scratch_types=[
            pltpu.VMEM((MAX_LEN, 128), jnp.int32),
            pltpu.VMEM((128, _W), jnp.float32),
            pltpu.VMEM((128, _W), jnp.float32),
            pltpu.VMEM((MAX_LEN, EMB), jnp.float32),
            pltpu.VMEM((8, 8, 129), jnp.float32),
            pltpu.VMEM((8, 8, 129), jnp.float32),
            pltpu.SemaphoreType.DMA,
            pltpu.SemaphoreType.DMA,
            pltpu.SemaphoreType.DMA,
            pltpu.SemaphoreType.DMA,
        ],
        compiler_params=pltpu.CompilerParams(
            use_tc_tiling_on_sc=False, needs_layout_passes=False),
    )
    def k2(tab_hbm, seqsT_hbm, pos_hbm, out_hbm,
           idxv, rows_a, rows_b, posv, til_a, til_b,
           gsa, gsb, osa, osb):
        wid = lax.axis_index("s") * _NC + lax.axis_index("c")
        rsv = []
        for q in range(EMB // _L):
            d = lax.iota(jnp.int32, _L) + _L * q
            rsv.append((d >> 3, d & 7))

        pltpu.sync_copy(seqsT_hbm.at[:, pl.ds(wid * 128, 128)], idxv)
        pltpu.sync_copy(pos_hbm, posv)

        def fetch_start(l, rows, gsem):
            pltpu.async_copy(tab_hbm.at[idxv.at[l]], rows, gsem)

        def fetch_wait(l, rows, gsem):
            pltpu.make_async_copy(tab_hbm.at[idxv.at[l]], rows, gsem).wait()

        def out_start(l, tiles, sem):
            pltpu.async_copy(tiles.at[:, :, pl.ds(0, 128)],
                             out_hbm.at[l, :, wid], sem)

        def out_wait(l, tiles, sem):
            pltpu.make_async_copy(tiles.at[:, :, pl.ds(0, 128)],
                                  out_hbm.at[l, :, wid], sem).wait()

        def compute(l, rows, tiles):
            pos4 = [posv[l, pl.ds(_L * q, _L)] for q in range(EMB // _L)]

            @plsc.parallel_loop(0, 128, unroll=4)
            def cbody(lane):
                lv = jnp.full((_L,), lane, jnp.int32)
                for q in range(EMB // _L):
                    v = rows[lane, pl.ds(_L * q, _L)] + pos4[q]
                    plsc.store_scatter(tiles, [rsv[q][0], rsv[q][1], lv], v)

        fetch_start(0, rows_a, gsa)

        def body(i, carry):
            l0 = 2 * i
            l1 = l0 + 1

            @pl.when(i > 0)
            def _():
                out_wait(l1 - 2, til_b, osb)
            fetch_start(l1, rows_b, gsb)

            fetch_wait(l0, rows_a, gsa)
            compute(l0, rows_a, til_a)
            out_start(l0, til_a, osa)

            @pl.when(i < MAX_LEN // 2 - 1)
            def _():
                out_wait(l0, til_a, osa)
                fetch_start(l0 + 2, rows_a, gsa)

            fetch_wait(l1, rows_b, gsb)
            compute(l1, rows_b, til_b)
            out_start(l1, til_b, osb)
            return carry

        lax.fori_loop(0, MAX_LEN // 2, body, 0)
        out_wait(MAX_LEN - 2, til_a, osa)
        out_wait(MAX_LEN - 1, til_b, osb)

    return k2


_K1 = _make_k1()
_K2 = _make_k2()


def kernel(batch_seqs, token_table, pos_table):
    seqsT = batch_seqs.astype(jnp.int32).T
    lin = _K1(token_table.T)
    out5 = _K2(lin.reshape(_PAD_ITEMS, _W), seqsT, pos_table)
    return out5.transpose(2, 4, 0, 1, 3).reshape(BATCH, MAX_LEN, EMB)
